# baseline (speedup 1.0000x reference)
import functools

import jax
import jax.numpy as jnp
from jax import lax
from jax.experimental import pallas as pl
from jax.experimental.pallas import tpu as pltpu
from jax.experimental.pallas import tpu_sc as plsc

N_NODES = 10000
NP = 10240
N_EDGES = 320000
CHUNK = 112
N_SUBCORES = 16
N_CORES = 2
EP = 32 * CHUNK * 93
CH_PER_SUB_1 = EP // (N_SUBCORES * CHUNK)
CH_PER_SUB_2 = EP // (N_CORES * N_SUBCORES * CHUNK)
ROWS_PER_SUB = NP // N_SUBCORES


def _hamilton(W1):
    r, i, j, k = jnp.split(W1, 4, axis=1)
    r2 = jnp.concatenate([r, -i, -j, -k], axis=0)
    i2 = jnp.concatenate([i, r, -k, j], axis=0)
    j2 = jnp.concatenate([j, k, r, -i], axis=0)
    k2 = jnp.concatenate([k, -j, i, r], axis=0)
    return jnp.concatenate([r2, i2, j2, k2], axis=1)


def _mm_body(a_ref, b_ref, o_ref):
    o_ref[...] = jnp.dot(a_ref[...], b_ref[...],
                         preferred_element_type=jnp.float32,
                         precision=lax.Precision.HIGHEST)


def _matmul(a, b, block_rows=2000):
    m, k = a.shape
    _, n = b.shape
    return pl.pallas_call(
        _mm_body,
        grid=(m // block_rows,),
        in_specs=[
            pl.BlockSpec((block_rows, k), lambda i: (i, 0)),
            pl.BlockSpec((k, n), lambda i: (0, 0)),
        ],
        out_specs=pl.BlockSpec((block_rows, n), lambda i: (i, 0)),
        out_shape=jax.ShapeDtypeStruct((m, n), jnp.float32),
    )(a, b)


def _mm2_body(ha_ref, hb_ref, wa_ref, wb_ref, o_ref):
    ha = jnp.maximum(ha_ref[...], 0.0)
    hb = jnp.maximum(hb_ref[...], 0.0)
    o_ref[...] = (
        jnp.dot(ha, wa_ref[...], preferred_element_type=jnp.float32,
                precision=lax.Precision.HIGHEST)
        + jnp.dot(hb, wb_ref[...], preferred_element_type=jnp.float32,
                  precision=lax.Precision.HIGHEST)
    )


def _relu_matmul2(ha, hb, w2a, w2b, block_rows=2000):
    m, k = ha.shape
    _, n = w2a.shape
    return pl.pallas_call(
        _mm2_body,
        grid=(m // block_rows,),
        in_specs=[
            pl.BlockSpec((block_rows, k), lambda i: (i, 0)),
            pl.BlockSpec((block_rows, k), lambda i: (i, 0)),
            pl.BlockSpec((k, n), lambda i: (0, 0)),
            pl.BlockSpec((k, n), lambda i: (0, 0)),
        ],
        out_specs=pl.BlockSpec((block_rows, n), lambda i: (i, 0)),
        out_shape=jax.ShapeDtypeStruct((m, n), jnp.float32),
    )(ha, hb, w2a, w2b)


def _final_body(p0_ref, p1_ref, o_ref):
    o = p0_ref[...][:, :64] + p1_ref[...][:, :64]
    m = jnp.max(o, axis=1, keepdims=True)
    e = jnp.exp(o - m)
    s = jnp.sum(e, axis=1, keepdims=True)
    o_ref[...] = (o - m) - jnp.log(s)


def _add_log_softmax(p0, p1, block_rows=2000):
    n = p0.shape[1]
    return pl.pallas_call(
        _final_body,
        grid=(N_NODES // block_rows,),
        in_specs=[
            pl.BlockSpec((block_rows, n), lambda i: (i, 0)),
            pl.BlockSpec((block_rows, n), lambda i: (i, 0)),
        ],
        out_specs=pl.BlockSpec((block_rows, 64), lambda i: (i, 0)),
        out_shape=jax.ShapeDtypeStruct((N_NODES, 64), jnp.float32),
    )(p0, p1)


def _zero_spmem(acc, rows, s, width):
    zero16 = jnp.zeros((16,), jnp.float32)

    def zbody(r, carry):
        for k in range(width // 16):
            rows[r, pl.ds(k * 16, 16)] = zero16
        return carry

    lax.fori_loop(0, CHUNK, zbody, 0)
    for j in range(ROWS_PER_SUB // 80):
        pltpu.sync_copy(rows.at[pl.ds(0, 80)],
                        acc.at[pl.ds(s * ROWS_PER_SUB + j * 80, 80)])


def _scale_rows(rows, wbuf, width):
    def gbody(g, carry):
        w16 = wbuf[pl.ds(g * 16, 16)]
        for lane in range(16):
            wb = lax.broadcast(w16[lane], (16,))
            for k in range(width // 16):
                sl = pl.ds(k * 16, 16)
                rows[g * 16 + lane, sl] = rows[g * 16 + lane, sl] * wb
        return carry

    lax.fori_loop(0, CHUNK // 16, gbody, 0)


def _pipelined_edge_loop(table, epack, wpack, acc, nch, cid0, bufs, width):
    ebufs, wbufs, rowss, sidxs, gsems, ssems = bufs

    def gstart(b, ci):
        pltpu.async_copy(table.at[ebufs[b].at[0]], rowss[b], gsems[b])

    def gwait(b):
        pltpu.make_async_copy(table.at[ebufs[b].at[0]], rowss[b],
                              gsems[b]).wait()

    def sstart(b):
        pltpu.async_copy(rowss[b], acc.at[sidxs[b]], ssems[b], add=True)

    def swait(b):
        pltpu.make_async_copy(rowss[b], acc.at[sidxs[b]], ssems[b]).wait()

    def eload(b, ci):
        pltpu.sync_copy(epack.at[cid0 + ci], ebufs[b])
        pltpu.sync_copy(wpack.at[cid0 + ci], wbufs[b])

    eload(0, 0)
    gstart(0, 0)
    eload(1, 1)

    def triple(p, carry):
        for b in range(3):
            ci = 3 * p + b
            gwait(b)

            @pl.when(ci >= 2)
            def _():
                swait((b + 1) % 3)

            @pl.when(ci + 1 < nch)
            def _():
                gstart((b + 1) % 3, ci + 1)

            _scale_rows(rowss[b], wbufs[b], width)
            for k in range(CHUNK // 16):
                sidxs[b][pl.ds(k * 16, 16)] = ebufs[b][1, pl.ds(k * 16, 16)]
            sstart(b)

            @pl.when(ci + 2 < nch)
            def _():
                eload((b + 2) % 3, ci + 2)
        return carry

    lax.fori_loop(0, nch // 3, triple, 0)
    swait(1)
    swait(2)


_SPMM_SCRATCH = [
    pltpu.VMEM((2, CHUNK), jnp.int32),
    pltpu.VMEM((2, CHUNK), jnp.int32),
    pltpu.VMEM((2, CHUNK), jnp.int32),
    pltpu.VMEM((CHUNK,), jnp.float32),
    pltpu.VMEM((CHUNK,), jnp.float32),
    pltpu.VMEM((CHUNK,), jnp.float32),
    pltpu.VMEM((CHUNK, 128), jnp.float32),
    pltpu.VMEM((CHUNK, 128), jnp.float32),
    pltpu.VMEM((CHUNK, 128), jnp.float32),
    pltpu.VMEM((CHUNK,), jnp.int32),
    pltpu.VMEM((CHUNK,), jnp.int32),
    pltpu.VMEM((CHUNK,), jnp.int32),
    pltpu.SemaphoreType.DMA,
    pltpu.SemaphoreType.DMA,
    pltpu.SemaphoreType.DMA,
    pltpu.SemaphoreType.DMA,
    pltpu.SemaphoreType.DMA,
    pltpu.SemaphoreType.DMA,
]


def _make_spmm1():
    mesh = plsc.VectorSubcoreMesh(core_axis_name="c", subcore_axis_name="s")

    @functools.partial(
        pl.kernel,
        mesh=mesh,
        out_type=[
            jax.ShapeDtypeStruct((NP, 128), jnp.float32),
            jax.ShapeDtypeStruct((NP, 128), jnp.float32),
        ],
        scratch_types=_SPMM_SCRATCH + [
            pltpu.VMEM_SHARED((NP, 128), jnp.float32),
        ],
    )
    def spmm1(sup_a, sup_b, epack, wpack, out_a, out_b,
              e0, e1, e2, w0, w1, w2, r0, r1, r2, i0, i1, i2,
              g0, g1, g2, s0, s1, s2, acc):
        c = lax.axis_index("c")
        s = lax.axis_index("s")
        bufs = ((e0, e1, e2), (w0, w1, w2), (r0, r1, r2), (i0, i1, i2),
                (g0, g1, g2), (s0, s1, s2))

        _zero_spmem(acc, r0, s, 128)
        plsc.subcore_barrier()

        @pl.when(c == 0)
        def _():
            _pipelined_edge_loop(sup_a, epack, wpack, acc, CH_PER_SUB_1,
                                 s * CH_PER_SUB_1, bufs, 128)

        @pl.when(c == 1)
        def _():
            _pipelined_edge_loop(sup_b, epack, wpack, acc, CH_PER_SUB_1,
                                 s * CH_PER_SUB_1, bufs, 128)

        plsc.subcore_barrier()

        @pl.when(c == 0)
        def _():
            pltpu.sync_copy(acc.at[pl.ds(s * ROWS_PER_SUB, ROWS_PER_SUB)],
                            out_a.at[pl.ds(s * ROWS_PER_SUB, ROWS_PER_SUB)])

        @pl.when(c == 1)
        def _():
            pltpu.sync_copy(acc.at[pl.ds(s * ROWS_PER_SUB, ROWS_PER_SUB)],
                            out_b.at[pl.ds(s * ROWS_PER_SUB, ROWS_PER_SUB)])

    return spmm1


def _make_spmm2():
    mesh = plsc.VectorSubcoreMesh(core_axis_name="c", subcore_axis_name="s")

    @functools.partial(
        pl.kernel,
        mesh=mesh,
        out_type=[
            jax.ShapeDtypeStruct((NP, 128), jnp.float32),
            jax.ShapeDtypeStruct((NP, 128), jnp.float32),
        ],
        scratch_types=_SPMM_SCRATCH + [
            pltpu.VMEM_SHARED((NP, 128), jnp.float32),
        ],
    )
    def spmm2(sup, epack, wpack, out_p0, out_p1,
              e0, e1, e2, w0, w1, w2, r0, r1, r2, i0, i1, i2,
              g0, g1, g2, s0, s1, s2, acc):
        c = lax.axis_index("c")
        s = lax.axis_index("s")
        bufs = ((e0, e1, e2), (w0, w1, w2), (r0, r1, r2), (i0, i1, i2),
                (g0, g1, g2), (s0, s1, s2))

        _zero_spmem(acc, r0, s, 128)
        plsc.subcore_barrier()

        wid = c * N_SUBCORES + s
        _pipelined_edge_loop(sup, epack, wpack, acc, CH_PER_SUB_2,
                             wid * CH_PER_SUB_2, bufs, 64)

        plsc.subcore_barrier()

        @pl.when(c == 0)
        def _():
            pltpu.sync_copy(acc.at[pl.ds(s * ROWS_PER_SUB, ROWS_PER_SUB)],
                            out_p0.at[pl.ds(s * ROWS_PER_SUB, ROWS_PER_SUB)])

        @pl.when(c == 1)
        def _():
            pltpu.sync_copy(acc.at[pl.ds(s * ROWS_PER_SUB, ROWS_PER_SUB)],
                            out_p1.at[pl.ds(s * ROWS_PER_SUB, ROWS_PER_SUB)])

    return spmm2


def kernel(x, edge_index, edge_weight, W1, W2):
    ham = _hamilton(W1)

    pad = EP - N_EDGES
    row = jnp.concatenate(
        [edge_index[0].astype(jnp.int32), jnp.zeros((pad,), jnp.int32)])
    col = jnp.concatenate(
        [edge_index[1].astype(jnp.int32), jnp.zeros((pad,), jnp.int32)])
    w = jnp.concatenate([edge_weight, jnp.zeros((pad,), jnp.float32)])
    epack = jnp.stack([col.reshape(-1, CHUNK), row.reshape(-1, CHUNK)], axis=1)
    wpack = w.reshape(-1, CHUNK)

    support_a = _matmul(x, ham[:, :128])
    support_b = _matmul(x, ham[:, 128:])

    h_a, h_b = _make_spmm1()(support_a, support_b, epack, wpack)

    w2p = jnp.concatenate([W2, jnp.zeros((W2.shape[0], 64), jnp.float32)], axis=1)
    s2 = _relu_matmul2(h_a, h_b, w2p[:128], w2p[128:], block_rows=1024)

    p0, p1 = _make_spmm2()(s2, epack, wpack)

    return _add_log_softmax(p0, p1)


# speedup vs baseline: 1.0005x; 1.0005x over previous
import functools

import jax
import jax.numpy as jnp
from jax import lax
from jax.experimental import pallas as pl
from jax.experimental.pallas import tpu as pltpu
from jax.experimental.pallas import tpu_sc as plsc

N_NODES = 10000
NP = 10240
N_EDGES = 320000
CHUNK = 112
N_SUBCORES = 16
N_CORES = 2
EP = 32 * CHUNK * 93
CH_PER_SUB_1 = EP // (N_SUBCORES * CHUNK)
CH_PER_SUB_2 = EP // (N_CORES * N_SUBCORES * CHUNK)
ROWS_PER_SUB = NP // N_SUBCORES


def _hamilton(W1):
    r, i, j, k = jnp.split(W1, 4, axis=1)
    r2 = jnp.concatenate([r, -i, -j, -k], axis=0)
    i2 = jnp.concatenate([i, r, -k, j], axis=0)
    j2 = jnp.concatenate([j, k, r, -i], axis=0)
    k2 = jnp.concatenate([k, -j, i, r], axis=0)
    return jnp.concatenate([r2, i2, j2, k2], axis=1)


def _mm_body(a_ref, b_ref, o_ref):
    o_ref[...] = jnp.dot(a_ref[...], b_ref[...],
                         preferred_element_type=jnp.float32,
                         precision=lax.Precision.HIGHEST)


def _matmul(a, b, block_rows=2000):
    m, k = a.shape
    _, n = b.shape
    return pl.pallas_call(
        _mm_body,
        grid=(m // block_rows,),
        in_specs=[
            pl.BlockSpec((block_rows, k), lambda i: (i, 0)),
            pl.BlockSpec((k, n), lambda i: (0, 0)),
        ],
        out_specs=pl.BlockSpec((block_rows, n), lambda i: (i, 0)),
        out_shape=jax.ShapeDtypeStruct((m, n), jnp.float32),
    )(a, b)


def _mm2_body(ha_ref, hb_ref, wa_ref, wb_ref, o_ref):
    ha = jnp.maximum(ha_ref[...], 0.0)
    hb = jnp.maximum(hb_ref[...], 0.0)
    o_ref[...] = (
        jnp.dot(ha, wa_ref[...], preferred_element_type=jnp.float32,
                precision=lax.Precision.HIGHEST)
        + jnp.dot(hb, wb_ref[...], preferred_element_type=jnp.float32,
                  precision=lax.Precision.HIGHEST)
    )


def _relu_matmul2(ha, hb, w2a, w2b, block_rows=2000):
    m, k = ha.shape
    _, n = w2a.shape
    return pl.pallas_call(
        _mm2_body,
        grid=(m // block_rows,),
        in_specs=[
            pl.BlockSpec((block_rows, k), lambda i: (i, 0)),
            pl.BlockSpec((block_rows, k), lambda i: (i, 0)),
            pl.BlockSpec((k, n), lambda i: (0, 0)),
            pl.BlockSpec((k, n), lambda i: (0, 0)),
        ],
        out_specs=pl.BlockSpec((block_rows, n), lambda i: (i, 0)),
        out_shape=jax.ShapeDtypeStruct((m, n), jnp.float32),
    )(ha, hb, w2a, w2b)


def _final_body(p0_ref, p1_ref, o_ref):
    o = p0_ref[...][:, :64] + p1_ref[...][:, :64]
    m = jnp.max(o, axis=1, keepdims=True)
    e = jnp.exp(o - m)
    s = jnp.sum(e, axis=1, keepdims=True)
    o_ref[...] = (o - m) - jnp.log(s)


def _add_log_softmax(p0, p1, block_rows=2000):
    n = p0.shape[1]
    return pl.pallas_call(
        _final_body,
        grid=(N_NODES // block_rows,),
        in_specs=[
            pl.BlockSpec((block_rows, n), lambda i: (i, 0)),
            pl.BlockSpec((block_rows, n), lambda i: (i, 0)),
        ],
        out_specs=pl.BlockSpec((block_rows, 64), lambda i: (i, 0)),
        out_shape=jax.ShapeDtypeStruct((N_NODES, 64), jnp.float32),
    )(p0, p1)


def _zero_spmem(acc, rows, s, width):
    zero16 = jnp.zeros((16,), jnp.float32)

    def zbody(r, carry):
        for k in range(width // 16):
            rows[r, pl.ds(k * 16, 16)] = zero16
        return carry

    lax.fori_loop(0, CHUNK, zbody, 0)
    for j in range(ROWS_PER_SUB // 80):
        pltpu.sync_copy(rows.at[pl.ds(0, 80)],
                        acc.at[pl.ds(s * ROWS_PER_SUB + j * 80, 80)])


def _scale_rows(rows, wbuf, width):
    def gbody(g, carry):
        w16 = wbuf[pl.ds(g * 16, 16)]
        for lane in range(16):
            wb = lax.broadcast(w16[lane], (16,))
            for k in range(width // 16):
                sl = pl.ds(k * 16, 16)
                rows[g * 16 + lane, sl] = rows[g * 16 + lane, sl] * wb
        return carry

    lax.fori_loop(0, CHUNK // 16, gbody, 0)


def _pipelined_edge_loop(table, epack, wpack, acc, nch, cid0, bufs, width):
    ebufs, wbufs, rowss, sidxs, gsems, ssems = bufs

    def gstart(b, ci):
        pltpu.async_copy(table.at[ebufs[b].at[0]], rowss[b], gsems[b])

    def gwait(b):
        pltpu.make_async_copy(table.at[ebufs[b].at[0]], rowss[b],
                              gsems[b]).wait()

    def sstart(b):
        pltpu.async_copy(rowss[b], acc.at[sidxs[b]], ssems[b], add=True)

    def swait(b):
        pltpu.make_async_copy(rowss[b], acc.at[sidxs[b]], ssems[b]).wait()

    def eload(b, ci):
        pltpu.sync_copy(epack.at[cid0 + ci], ebufs[b])
        pltpu.sync_copy(wpack.at[cid0 + ci], wbufs[b])

    eload(0, 0)
    gstart(0, 0)
    eload(1, 1)

    def triple(p, carry):
        for b in range(3):
            ci = 3 * p + b
            gwait(b)

            @pl.when(ci >= 2)
            def _():
                swait((b + 1) % 3)

            @pl.when(ci + 1 < nch)
            def _():
                gstart((b + 1) % 3, ci + 1)

            _scale_rows(rowss[b], wbufs[b], width)
            for k in range(CHUNK // 16):
                sidxs[b][pl.ds(k * 16, 16)] = ebufs[b][1, pl.ds(k * 16, 16)]
            sstart(b)

            @pl.when(ci + 2 < nch)
            def _():
                eload((b + 2) % 3, ci + 2)
        return carry

    lax.fori_loop(0, nch // 3, triple, 0)
    swait(1)
    swait(2)


_SPMM_SCRATCH = [
    pltpu.VMEM((2, CHUNK), jnp.int32),
    pltpu.VMEM((2, CHUNK), jnp.int32),
    pltpu.VMEM((2, CHUNK), jnp.int32),
    pltpu.VMEM((CHUNK,), jnp.float32),
    pltpu.VMEM((CHUNK,), jnp.float32),
    pltpu.VMEM((CHUNK,), jnp.float32),
    pltpu.VMEM((CHUNK, 128), jnp.float32),
    pltpu.VMEM((CHUNK, 128), jnp.float32),
    pltpu.VMEM((CHUNK, 128), jnp.float32),
    pltpu.VMEM((CHUNK,), jnp.int32),
    pltpu.VMEM((CHUNK,), jnp.int32),
    pltpu.VMEM((CHUNK,), jnp.int32),
    pltpu.SemaphoreType.DMA,
    pltpu.SemaphoreType.DMA,
    pltpu.SemaphoreType.DMA,
    pltpu.SemaphoreType.DMA,
    pltpu.SemaphoreType.DMA,
    pltpu.SemaphoreType.DMA,
]


def _make_spmm1():
    mesh = plsc.VectorSubcoreMesh(core_axis_name="c", subcore_axis_name="s")

    @functools.partial(
        pl.kernel,
        mesh=mesh,
        out_type=[
            jax.ShapeDtypeStruct((NP, 128), jnp.float32),
            jax.ShapeDtypeStruct((NP, 128), jnp.float32),
        ],
        scratch_types=_SPMM_SCRATCH + [
            pltpu.VMEM_SHARED((NP, 128), jnp.float32),
        ],
    )
    def spmm1(sup_a, sup_b, epack, wpack, out_a, out_b,
              e0, e1, e2, w0, w1, w2, r0, r1, r2, i0, i1, i2,
              g0, g1, g2, s0, s1, s2, acc):
        c = lax.axis_index("c")
        s = lax.axis_index("s")
        bufs = ((e0, e1, e2), (w0, w1, w2), (r0, r1, r2), (i0, i1, i2),
                (g0, g1, g2), (s0, s1, s2))

        _zero_spmem(acc, r0, s, 128)
        plsc.subcore_barrier()

        @pl.when(c == 0)
        def _():
            _pipelined_edge_loop(sup_a, epack, wpack, acc, CH_PER_SUB_1,
                                 s * CH_PER_SUB_1, bufs, 128)

        @pl.when(c == 1)
        def _():
            _pipelined_edge_loop(sup_b, epack, wpack, acc, CH_PER_SUB_1,
                                 s * CH_PER_SUB_1, bufs, 128)

        plsc.subcore_barrier()

        @pl.when(c == 0)
        def _():
            pltpu.sync_copy(acc.at[pl.ds(s * ROWS_PER_SUB, ROWS_PER_SUB)],
                            out_a.at[pl.ds(s * ROWS_PER_SUB, ROWS_PER_SUB)])

        @pl.when(c == 1)
        def _():
            pltpu.sync_copy(acc.at[pl.ds(s * ROWS_PER_SUB, ROWS_PER_SUB)],
                            out_b.at[pl.ds(s * ROWS_PER_SUB, ROWS_PER_SUB)])

    return spmm1


def _make_spmm2():
    mesh = plsc.VectorSubcoreMesh(core_axis_name="c", subcore_axis_name="s")

    @functools.partial(
        pl.kernel,
        mesh=mesh,
        out_type=[
            jax.ShapeDtypeStruct((NP, 128), jnp.float32),
            jax.ShapeDtypeStruct((NP, 128), jnp.float32),
        ],
        scratch_types=_SPMM_SCRATCH + [
            pltpu.VMEM_SHARED((NP, 128), jnp.float32),
        ],
    )
    def spmm2(sup, epack, wpack, out_p0, out_p1,
              e0, e1, e2, w0, w1, w2, r0, r1, r2, i0, i1, i2,
              g0, g1, g2, s0, s1, s2, acc):
        c = lax.axis_index("c")
        s = lax.axis_index("s")
        bufs = ((e0, e1, e2), (w0, w1, w2), (r0, r1, r2), (i0, i1, i2),
                (g0, g1, g2), (s0, s1, s2))

        _zero_spmem(acc, r0, s, 128)
        plsc.subcore_barrier()

        wid = c * N_SUBCORES + s
        _pipelined_edge_loop(sup, epack, wpack, acc, CH_PER_SUB_2,
                             wid * CH_PER_SUB_2, bufs, 64)

        plsc.subcore_barrier()

        @pl.when(c == 0)
        def _():
            pltpu.sync_copy(acc.at[pl.ds(s * ROWS_PER_SUB, ROWS_PER_SUB)],
                            out_p0.at[pl.ds(s * ROWS_PER_SUB, ROWS_PER_SUB)])

        @pl.when(c == 1)
        def _():
            pltpu.sync_copy(acc.at[pl.ds(s * ROWS_PER_SUB, ROWS_PER_SUB)],
                            out_p1.at[pl.ds(s * ROWS_PER_SUB, ROWS_PER_SUB)])

    return spmm2


def kernel(x, edge_index, edge_weight, W1, W2):
    ham = _hamilton(W1)

    pad = EP - N_EDGES
    pad_rows = N_NODES + (jnp.arange(pad, dtype=jnp.int32) % (NP - N_NODES))
    row = jnp.concatenate(
        [edge_index[0].astype(jnp.int32), pad_rows])
    col = jnp.concatenate(
        [edge_index[1].astype(jnp.int32), jnp.zeros((pad,), jnp.int32)])
    w = jnp.concatenate([edge_weight, jnp.zeros((pad,), jnp.float32)])
    epack = jnp.stack([col.reshape(-1, CHUNK), row.reshape(-1, CHUNK)], axis=1)
    wpack = w.reshape(-1, CHUNK)

    support_a = _matmul(x, ham[:, :128])
    support_b = _matmul(x, ham[:, 128:])

    h_a, h_b = _make_spmm1()(support_a, support_b, epack, wpack)

    w2p = jnp.concatenate([W2, jnp.zeros((W2.shape[0], 64), jnp.float32)], axis=1)
    s2 = _relu_matmul2(h_a, h_b, w2p[:128], w2p[128:], block_rows=1024)

    p0, p1 = _make_spmm2()(s2, epack, wpack)

    return _add_log_softmax(p0, p1)


# speedup vs baseline: 4.3503x; 4.3480x over previous
import functools

import jax
import jax.numpy as jnp
from jax import lax
from jax.experimental import pallas as pl
from jax.experimental.pallas import tpu as pltpu
from jax.experimental.pallas import tpu_sc as plsc

N_NODES = 10000
NP = 10240
N_EDGES = 320000
CHUNK = 112
N_SUBCORES = 16
N_CORES = 2
EP = 32 * CHUNK * 93
CH_PER_SUB = EP // (N_CORES * N_SUBCORES * CHUNK)
ROWS_PER_SUB = NP // N_SUBCORES


def _hamilton(W1):
    r, i, j, k = jnp.split(W1, 4, axis=1)
    r2 = jnp.concatenate([r, -i, -j, -k], axis=0)
    i2 = jnp.concatenate([i, r, -k, j], axis=0)
    j2 = jnp.concatenate([j, k, r, -i], axis=0)
    k2 = jnp.concatenate([k, -j, i, r], axis=0)
    return jnp.concatenate([r2, i2, j2, k2], axis=1)


def _ffn_body(g0_ref, g1_ref, ham_ref, w2p_ref, o_ref):
    g = g0_ref[...] + g1_ref[...]
    h = jnp.maximum(jnp.dot(g, ham_ref[...],
                            preferred_element_type=jnp.float32,
                            precision=lax.Precision.HIGHEST), 0.0)
    o_ref[...] = jnp.dot(h, w2p_ref[...],
                         preferred_element_type=jnp.float32,
                         precision=lax.Precision.HIGHEST)


def _ffn(g0, g1, ham, w2p, block_rows=1024):
    m = g0.shape[0]
    return pl.pallas_call(
        _ffn_body,
        grid=(m // block_rows,),
        in_specs=[
            pl.BlockSpec((block_rows, 128), lambda i: (i, 0)),
            pl.BlockSpec((block_rows, 128), lambda i: (i, 0)),
            pl.BlockSpec((128, 256), lambda i: (0, 0)),
            pl.BlockSpec((256, 128), lambda i: (0, 0)),
        ],
        out_specs=pl.BlockSpec((block_rows, 128), lambda i: (i, 0)),
        out_shape=jax.ShapeDtypeStruct((m, 128), jnp.float32),
    )(g0, g1, ham, w2p)


def _final_body(p0_ref, p1_ref, o_ref):
    o = p0_ref[...][:, :64] + p1_ref[...][:, :64]
    m = jnp.max(o, axis=1, keepdims=True)
    e = jnp.exp(o - m)
    s = jnp.sum(e, axis=1, keepdims=True)
    o_ref[...] = (o - m) - jnp.log(s)


def _add_log_softmax(p0, p1, block_rows=2000):
    n = p0.shape[1]
    return pl.pallas_call(
        _final_body,
        grid=(N_NODES // block_rows,),
        in_specs=[
            pl.BlockSpec((block_rows, n), lambda i: (i, 0)),
            pl.BlockSpec((block_rows, n), lambda i: (i, 0)),
        ],
        out_specs=pl.BlockSpec((block_rows, 64), lambda i: (i, 0)),
        out_shape=jax.ShapeDtypeStruct((N_NODES, 64), jnp.float32),
    )(p0, p1)


def _zero_spmem(acc, rows, s, width):
    zero16 = jnp.zeros((16,), jnp.float32)

    def zbody(r, carry):
        for k in range(width // 16):
            rows[r, pl.ds(k * 16, 16)] = zero16
        return carry

    lax.fori_loop(0, CHUNK, zbody, 0)
    for j in range(ROWS_PER_SUB // 80):
        pltpu.sync_copy(rows.at[pl.ds(0, 80)],
                        acc.at[pl.ds(s * ROWS_PER_SUB + j * 80, 80)])


def _scale_rows(rows, wbuf, width):
    def gbody(g, carry):
        w16 = wbuf[pl.ds(g * 16, 16)]
        for lane in range(16):
            wb = lax.broadcast(w16[lane], (16,))
            for k in range(width // 16):
                sl = pl.ds(k * 16, 16)
                rows[g * 16 + lane, sl] = rows[g * 16 + lane, sl] * wb
        return carry

    lax.fori_loop(0, CHUNK // 16, gbody, 0)


def _pipelined_edge_loop(table, epack, wpack, acc, nch, cid0, bufs, width):
    ebufs, wbufs, rowss, sidxs, gsems, ssems = bufs

    def gstart(b, ci):
        pltpu.async_copy(table.at[ebufs[b].at[0]], rowss[b], gsems[b])

    def gwait(b):
        pltpu.make_async_copy(table.at[ebufs[b].at[0]], rowss[b],
                              gsems[b]).wait()

    def sstart(b):
        pltpu.async_copy(rowss[b], acc.at[sidxs[b]], ssems[b], add=True)

    def swait(b):
        pltpu.make_async_copy(rowss[b], acc.at[sidxs[b]], ssems[b]).wait()

    def eload(b, ci):
        pltpu.sync_copy(epack.at[cid0 + ci], ebufs[b])
        pltpu.sync_copy(wpack.at[cid0 + ci], wbufs[b])

    eload(0, 0)
    gstart(0, 0)
    eload(1, 1)

    def triple(p, carry):
        for b in range(3):
            ci = 3 * p + b
            gwait(b)

            @pl.when(ci >= 2)
            def _():
                swait((b + 1) % 3)

            @pl.when(ci + 1 < nch)
            def _():
                gstart((b + 1) % 3, ci + 1)

            _scale_rows(rowss[b], wbufs[b], width)
            for k in range(CHUNK // 16):
                sidxs[b][pl.ds(k * 16, 16)] = ebufs[b][1, pl.ds(k * 16, 16)]
            sstart(b)

            @pl.when(ci + 2 < nch)
            def _():
                eload((b + 2) % 3, ci + 2)
        return carry

    lax.fori_loop(0, nch // 3, triple, 0)
    swait(1)
    swait(2)


_SPMM_SCRATCH = [
    pltpu.VMEM((2, CHUNK), jnp.int32),
    pltpu.VMEM((2, CHUNK), jnp.int32),
    pltpu.VMEM((2, CHUNK), jnp.int32),
    pltpu.VMEM((CHUNK,), jnp.float32),
    pltpu.VMEM((CHUNK,), jnp.float32),
    pltpu.VMEM((CHUNK,), jnp.float32),
    pltpu.VMEM((CHUNK, 128), jnp.float32),
    pltpu.VMEM((CHUNK, 128), jnp.float32),
    pltpu.VMEM((CHUNK, 128), jnp.float32),
    pltpu.VMEM((CHUNK,), jnp.int32),
    pltpu.VMEM((CHUNK,), jnp.int32),
    pltpu.VMEM((CHUNK,), jnp.int32),
    pltpu.SemaphoreType.DMA,
    pltpu.SemaphoreType.DMA,
    pltpu.SemaphoreType.DMA,
    pltpu.SemaphoreType.DMA,
    pltpu.SemaphoreType.DMA,
    pltpu.SemaphoreType.DMA,
]


def _make_spmm(table_shape, scale_width):
    mesh = plsc.VectorSubcoreMesh(core_axis_name="c", subcore_axis_name="s")

    @functools.partial(
        pl.kernel,
        mesh=mesh,
        out_type=[
            jax.ShapeDtypeStruct((NP, 128), jnp.float32),
            jax.ShapeDtypeStruct((NP, 128), jnp.float32),
        ],
        scratch_types=_SPMM_SCRATCH + [
            pltpu.VMEM_SHARED((NP, 128), jnp.float32),
        ],
    )
    def spmm(sup, epack, wpack, out_p0, out_p1,
             e0, e1, e2, w0, w1, w2, r0, r1, r2, i0, i1, i2,
             g0, g1, g2, s0, s1, s2, acc):
        c = lax.axis_index("c")
        s = lax.axis_index("s")
        bufs = ((e0, e1, e2), (w0, w1, w2), (r0, r1, r2), (i0, i1, i2),
                (g0, g1, g2), (s0, s1, s2))

        _zero_spmem(acc, r0, s, 128)
        plsc.subcore_barrier()

        wid = c * N_SUBCORES + s
        _pipelined_edge_loop(sup, epack, wpack, acc, CH_PER_SUB,
                             wid * CH_PER_SUB, bufs, scale_width)

        plsc.subcore_barrier()

        @pl.when(c == 0)
        def _():
            pltpu.sync_copy(acc.at[pl.ds(s * ROWS_PER_SUB, ROWS_PER_SUB)],
                            out_p0.at[pl.ds(s * ROWS_PER_SUB, ROWS_PER_SUB)])

        @pl.when(c == 1)
        def _():
            pltpu.sync_copy(acc.at[pl.ds(s * ROWS_PER_SUB, ROWS_PER_SUB)],
                            out_p1.at[pl.ds(s * ROWS_PER_SUB, ROWS_PER_SUB)])

    return spmm


def kernel(x, edge_index, edge_weight, W1, W2):
    ham = _hamilton(W1)

    pad = EP - N_EDGES
    pad_rows = N_NODES + (jnp.arange(pad, dtype=jnp.int32) % (NP - N_NODES))
    row = jnp.concatenate(
        [edge_index[0].astype(jnp.int32), pad_rows])
    pad_cols = jnp.arange(pad, dtype=jnp.int32) % N_NODES
    col = jnp.concatenate(
        [edge_index[1].astype(jnp.int32), pad_cols])
    w = jnp.concatenate([edge_weight, jnp.zeros((pad,), jnp.float32)])
    epack = jnp.stack([col.reshape(-1, CHUNK), row.reshape(-1, CHUNK)], axis=1)
    wpack = w.reshape(-1, CHUNK)

    g0, g1 = _make_spmm(x.shape, 128)(x, epack, wpack)

    w2p = jnp.concatenate([W2, jnp.zeros((W2.shape[0], 64), jnp.float32)],
                          axis=1)
    s2 = _ffn(g0, g1, ham, w2p)

    p0, p1 = _make_spmm(s2.shape, 64)(s2, epack, wpack)

    return _add_log_softmax(p0, p1)


# speedup vs baseline: 4.7192x; 1.0848x over previous
import functools

import jax
import jax.numpy as jnp
from jax import lax
from jax.experimental import pallas as pl
from jax.experimental.pallas import tpu as pltpu
from jax.experimental.pallas import tpu_sc as plsc

N_NODES = 10000
NP = 10240
N_EDGES = 320000
CHUNK = 112
N_SUBCORES = 16
N_CORES = 2
EP = 32 * CHUNK * 93
CH_PER_SUB = EP // (N_CORES * N_SUBCORES * CHUNK)
ROWS_PER_SUB = NP // N_SUBCORES


def _hamilton(W1):
    r, i, j, k = jnp.split(W1, 4, axis=1)
    r2 = jnp.concatenate([r, -i, -j, -k], axis=0)
    i2 = jnp.concatenate([i, r, -k, j], axis=0)
    j2 = jnp.concatenate([j, k, r, -i], axis=0)
    k2 = jnp.concatenate([k, -j, i, r], axis=0)
    return jnp.concatenate([r2, i2, j2, k2], axis=1)


def _ffn_body(g0_ref, g1_ref, ham_ref, w2p_ref, o_ref):
    g = g0_ref[...] + g1_ref[...]
    h = jnp.maximum(jnp.dot(g, ham_ref[...],
                            preferred_element_type=jnp.float32,
                            precision=lax.Precision.HIGHEST), 0.0)
    o_ref[...] = jnp.dot(h, w2p_ref[...],
                         preferred_element_type=jnp.float32,
                         precision=lax.Precision.HIGHEST)


def _ffn(g0, g1, ham, w2p, block_rows=1024):
    m = g0.shape[0]
    return pl.pallas_call(
        _ffn_body,
        grid=(m // block_rows,),
        in_specs=[
            pl.BlockSpec((block_rows, 128), lambda i: (i, 0)),
            pl.BlockSpec((block_rows, 128), lambda i: (i, 0)),
            pl.BlockSpec((128, 256), lambda i: (0, 0)),
            pl.BlockSpec((256, 128), lambda i: (0, 0)),
        ],
        out_specs=pl.BlockSpec((block_rows, 128), lambda i: (i, 0)),
        out_shape=jax.ShapeDtypeStruct((m, 128), jnp.float32),
    )(g0, g1, ham, w2p)


def _final_body(p0_ref, p1_ref, o_ref):
    o = p0_ref[...][:, :64] + p1_ref[...][:, :64]
    m = jnp.max(o, axis=1, keepdims=True)
    e = jnp.exp(o - m)
    s = jnp.sum(e, axis=1, keepdims=True)
    o_ref[...] = (o - m) - jnp.log(s)


def _add_log_softmax(p0, p1, block_rows=2000):
    n = p0.shape[1]
    return pl.pallas_call(
        _final_body,
        grid=(N_NODES // block_rows,),
        in_specs=[
            pl.BlockSpec((block_rows, n), lambda i: (i, 0)),
            pl.BlockSpec((block_rows, n), lambda i: (i, 0)),
        ],
        out_specs=pl.BlockSpec((block_rows, 64), lambda i: (i, 0)),
        out_shape=jax.ShapeDtypeStruct((N_NODES, 64), jnp.float32),
    )(p0, p1)


def _zero_spmem(acc, rows, s, width):
    zero16 = jnp.zeros((16,), jnp.float32)

    def zbody(r, carry):
        for k in range(width // 16):
            rows[r, pl.ds(k * 16, 16)] = zero16
        return carry

    lax.fori_loop(0, CHUNK, zbody, 0)
    for j in range(ROWS_PER_SUB // 80):
        pltpu.sync_copy(rows.at[pl.ds(0, 80)],
                        acc.at[pl.ds(s * ROWS_PER_SUB + j * 80, 80)])


def _scale_rows(rows, ebuf, width):
    def gbody(g, carry):
        w16 = ebuf[2, pl.ds(g * 16, 16)]
        for lane in range(16):
            wb = lax.broadcast(w16[lane], (16,))
            for k in range(width // 16):
                sl = pl.ds(k * 16, 16)
                rows[g * 16 + lane, sl] = rows[g * 16 + lane, sl] * wb
        return carry

    lax.fori_loop(0, CHUNK // 16, gbody, 0)


def _pipelined_edge_loop(table, epack, acc, nch, cid0, bufs, width):
    ebufs, rowss, gidx, sidxs, esems, gsems, ssems = bufs

    def eload(m, ci):
        pltpu.async_copy(epack.at[cid0 + ci], ebufs[m], esems[m])

    def ewait(m):
        pltpu.make_async_copy(epack.at[cid0], ebufs[m], esems[m]).wait()

    def conv(m):
        for k in range(CHUNK // 16):
            sl = pl.ds(k * 16, 16)
            gidx[sl] = ebufs[m][0, sl].astype(jnp.int32)
            sidxs[m][sl] = ebufs[m][1, sl].astype(jnp.int32)

    def gstart(m):
        pltpu.async_copy(table.at[gidx], rowss[m], gsems[m])

    def gwait(m):
        pltpu.make_async_copy(table.at[gidx], rowss[m], gsems[m]).wait()

    def sstart(m):
        pltpu.async_copy(rowss[m], acc.at[sidxs[m]], ssems[m], add=True)

    def swait(m):
        pltpu.make_async_copy(rowss[m], acc.at[sidxs[m]], ssems[m]).wait()

    eload(0, 0)
    ewait(0)
    conv(0)
    gstart(0)
    eload(1, 1)

    def triple(p, carry):
        for b in range(3):
            ci = 3 * p + b
            n = (b + 1) % 3
            gwait(b)

            @pl.when(ci + 1 < nch)
            def _():
                ewait(n)

            @pl.when(ci >= 2)
            def _():
                swait(n)

            @pl.when(ci + 1 < nch)
            def _():
                conv(n)
                gstart(n)

            @pl.when(ci + 2 < nch)
            def _():
                eload((b + 2) % 3, ci + 2)

            _scale_rows(rowss[b], ebufs[b], width)
            sstart(b)
        return carry

    lax.fori_loop(0, nch // 3, triple, 0)
    swait((nch - 2) % 3)
    swait((nch - 1) % 3)


_SPMM_SCRATCH = [
    pltpu.VMEM((3, CHUNK), jnp.float32),
    pltpu.VMEM((3, CHUNK), jnp.float32),
    pltpu.VMEM((3, CHUNK), jnp.float32),
    pltpu.VMEM((CHUNK, 128), jnp.float32),
    pltpu.VMEM((CHUNK, 128), jnp.float32),
    pltpu.VMEM((CHUNK, 128), jnp.float32),
    pltpu.VMEM((CHUNK,), jnp.int32),
    pltpu.VMEM((CHUNK,), jnp.int32),
    pltpu.VMEM((CHUNK,), jnp.int32),
    pltpu.VMEM((CHUNK,), jnp.int32),
    pltpu.SemaphoreType.DMA,
    pltpu.SemaphoreType.DMA,
    pltpu.SemaphoreType.DMA,
    pltpu.SemaphoreType.DMA,
    pltpu.SemaphoreType.DMA,
    pltpu.SemaphoreType.DMA,
    pltpu.SemaphoreType.DMA,
    pltpu.SemaphoreType.DMA,
    pltpu.SemaphoreType.DMA,
]


def _make_spmm(table_shape, scale_width):
    mesh = plsc.VectorSubcoreMesh(core_axis_name="c", subcore_axis_name="s")

    @functools.partial(
        pl.kernel,
        mesh=mesh,
        out_type=[
            jax.ShapeDtypeStruct((NP, 128), jnp.float32),
            jax.ShapeDtypeStruct((NP, 128), jnp.float32),
        ],
        scratch_types=_SPMM_SCRATCH + [
            pltpu.VMEM_SHARED((NP, 128), jnp.float32),
        ],
    )
    def spmm(sup, epack, out_p0, out_p1,
             e0, e1, e2, r0, r1, r2, gi, i0, i1, i2,
             es0, es1, es2, g0, g1, g2, s0, s1, s2, acc):
        c = lax.axis_index("c")
        s = lax.axis_index("s")
        bufs = ((e0, e1, e2), (r0, r1, r2), gi, (i0, i1, i2),
                (es0, es1, es2), (g0, g1, g2), (s0, s1, s2))

        _zero_spmem(acc, r0, s, 128)
        plsc.subcore_barrier()

        wid = c * N_SUBCORES + s
        _pipelined_edge_loop(sup, epack, acc, CH_PER_SUB,
                             wid * CH_PER_SUB, bufs, scale_width)

        plsc.subcore_barrier()

        @pl.when(c == 0)
        def _():
            pltpu.sync_copy(acc.at[pl.ds(s * ROWS_PER_SUB, ROWS_PER_SUB)],
                            out_p0.at[pl.ds(s * ROWS_PER_SUB, ROWS_PER_SUB)])

        @pl.when(c == 1)
        def _():
            pltpu.sync_copy(acc.at[pl.ds(s * ROWS_PER_SUB, ROWS_PER_SUB)],
                            out_p1.at[pl.ds(s * ROWS_PER_SUB, ROWS_PER_SUB)])

    return spmm


def kernel(x, edge_index, edge_weight, W1, W2):
    ham = _hamilton(W1)

    pad = EP - N_EDGES
    pad_rows = N_NODES + (jnp.arange(pad, dtype=jnp.int32) % (NP - N_NODES))
    row = jnp.concatenate(
        [edge_index[0].astype(jnp.int32), pad_rows])
    pad_cols = jnp.arange(pad, dtype=jnp.int32) % N_NODES
    col = jnp.concatenate(
        [edge_index[1].astype(jnp.int32), pad_cols])
    w = jnp.concatenate([edge_weight, jnp.zeros((pad,), jnp.float32)])
    epack = jnp.stack([col.astype(jnp.float32).reshape(-1, CHUNK),
                       row.astype(jnp.float32).reshape(-1, CHUNK),
                       w.reshape(-1, CHUNK)], axis=1)

    g0, g1 = _make_spmm(x.shape, 128)(x, epack)

    w2p = jnp.concatenate([W2, jnp.zeros((W2.shape[0], 64), jnp.float32)],
                          axis=1)
    s2 = _ffn(g0, g1, ham, w2p)

    p0, p1 = _make_spmm(s2.shape, 64)(s2, epack)

    return _add_log_softmax(p0, p1)
